# baseline (speedup 1.0000x reference)
import functools

import jax
import jax.numpy as jnp
from jax import lax
from jax.experimental import pallas as pl
from jax.experimental.pallas import tpu as pltpu
from jax.experimental.pallas import tpu_sc as plsc

NUM_EMB = 1000000
NUM_ROWS = 16384 * 26
DIM = 64
NC = 2
NS = 16
NW = NC * NS
ROWS_PER_W = NUM_ROWS // NW
CHUNK = 512
N_CHUNKS = ROWS_PER_W // CHUNK

_mesh = plsc.VectorSubcoreMesh(
    core_axis_name="c", subcore_axis_name="s", num_cores=NC, num_subcores=NS
)


@functools.partial(
    pl.kernel,
    out_type=jax.ShapeDtypeStruct((NUM_ROWS, DIM), jnp.float32),
    mesh=_mesh,
    scratch_types=[
        pltpu.VMEM((ROWS_PER_W,), jnp.int32),
        pltpu.VMEM((CHUNK, DIM), jnp.float32),
        pltpu.VMEM((CHUNK, DIM), jnp.float32),
        pltpu.SemaphoreType.DMA,
        pltpu.SemaphoreType.DMA,
    ],
    compiler_params=pltpu.CompilerParams(use_tc_tiling_on_sc=False),
)
def _gather_kernel(idx_hbm, table_hbm, out_hbm, idx_v, rows_a, rows_b, sem_a, sem_b):
    wid = lax.axis_index("s") * NC + lax.axis_index("c")
    base = wid * ROWS_PER_W
    pltpu.sync_copy(idx_hbm.at[pl.ds(base, ROWS_PER_W)], idx_v)

    bufs = (rows_a, rows_b)
    sems = (sem_a, sem_b)

    def start_gather(i):
        b = i % 2
        return pltpu.async_copy(
            table_hbm.at[idx_v.at[pl.ds(i * CHUNK, CHUNK)]], bufs[b], sems[b]
        )

    descs = {0: start_gather(0)}
    for i in range(N_CHUNKS):
        if i + 1 < N_CHUNKS:
            descs[i + 1] = start_gather(i + 1)
        descs[i].wait()
        pltpu.sync_copy(bufs[i % 2], out_hbm.at[pl.ds(base + i * CHUNK, CHUNK)])


def kernel(indices, embedding):
    flat_idx = indices.reshape(-1).astype(jnp.int32)
    out = _gather_kernel(flat_idx, embedding)
    return out.reshape(indices.shape + (DIM,))


# speedup vs baseline: 1.0734x; 1.0734x over previous
import functools

import jax
import jax.numpy as jnp
from jax import lax
from jax.experimental import pallas as pl
from jax.experimental.pallas import tpu as pltpu
from jax.experimental.pallas import tpu_sc as plsc

NUM_EMB = 1000000
NUM_ROWS = 16384 * 26
DIM = 64
NC = 2
NS = 16
NW = NC * NS
ROWS_PER_W = NUM_ROWS // NW
CHUNK = 512
N_CHUNKS = ROWS_PER_W // CHUNK

_mesh = plsc.VectorSubcoreMesh(
    core_axis_name="c", subcore_axis_name="s", num_cores=NC, num_subcores=NS
)


@functools.partial(
    pl.kernel,
    out_type=jax.ShapeDtypeStruct((NUM_ROWS, DIM), jnp.float32),
    mesh=_mesh,
    scratch_types=[
        pltpu.VMEM((ROWS_PER_W,), jnp.int32),
        pltpu.VMEM((CHUNK, DIM), jnp.float32),
        pltpu.VMEM((CHUNK, DIM), jnp.float32),
        pltpu.SemaphoreType.DMA,
        pltpu.SemaphoreType.DMA,
    ],
    compiler_params=pltpu.CompilerParams(use_tc_tiling_on_sc=False),
)
def _gather_kernel(idx_hbm, table_hbm, out_hbm, idx_v, rows_a, rows_b, sem_a, sem_b):
    wid = lax.axis_index("s") * NC + lax.axis_index("c")
    base = wid * ROWS_PER_W
    pltpu.sync_copy(idx_hbm.at[pl.ds(base, ROWS_PER_W)], idx_v)

    bufs = (rows_a, rows_b)
    sems = (sem_a, sem_b)

    def start_gather(i):
        b = i % 2
        return pltpu.async_copy(
            table_hbm.at[idx_v.at[pl.ds(i * CHUNK, CHUNK)]], bufs[b], sems[b]
        )

    descs = {0: start_gather(0)}
    for i in range(N_CHUNKS):
        if i + 1 < N_CHUNKS:
            descs[i + 1] = start_gather(i + 1)
        descs[i].wait()
        pltpu.sync_copy(bufs[i % 2], out_hbm.at[pl.ds(base + i * CHUNK, CHUNK)])


def kernel(indices, embedding):
    flat_idx = indices.reshape(-1).astype(jnp.int32) * 2
    emb_pad = jnp.pad(embedding, ((0, 0), (0, 128 - DIM)))
    emb_linear = emb_pad.reshape(2 * NUM_EMB, DIM)
    out = _gather_kernel(flat_idx, emb_linear)
    return out.reshape(indices.shape + (DIM,))


# speedup vs baseline: 1.1439x; 1.0657x over previous
import functools

import jax
import jax.numpy as jnp
from jax import lax
from jax.experimental import pallas as pl
from jax.experimental.pallas import tpu as pltpu
from jax.experimental.pallas import tpu_sc as plsc

NUM_EMB = 1000000
NUM_ROWS = 16384 * 26
DIM = 64
NC = 2
NS = 16
NW = NC * NS
ROWS_PER_W = NUM_ROWS // NW
CHUNK = 512
N_CHUNKS = ROWS_PER_W // CHUNK

_mesh = plsc.VectorSubcoreMesh(
    core_axis_name="c", subcore_axis_name="s", num_cores=NC, num_subcores=NS
)


@functools.partial(
    pl.kernel,
    out_type=jax.ShapeDtypeStruct((NUM_ROWS, DIM), jnp.float32),
    mesh=_mesh,
    scratch_types=[
        pltpu.VMEM((ROWS_PER_W,), jnp.int32),
        pltpu.VMEM((CHUNK, DIM), jnp.float32),
        pltpu.VMEM((CHUNK, DIM), jnp.float32),
        pltpu.SemaphoreType.DMA,
        pltpu.SemaphoreType.DMA,
    ],
    compiler_params=pltpu.CompilerParams(use_tc_tiling_on_sc=False),
)
def _gather_kernel(idx_hbm, table_hbm, out_hbm, idx_v, rows_a, rows_b, sem_a, sem_b):
    wid = lax.axis_index("s") * NC + lax.axis_index("c")
    base = wid * ROWS_PER_W
    pltpu.sync_copy(idx_hbm.at[pl.ds(base, ROWS_PER_W)], idx_v)

    bufs = (rows_a, rows_b)
    sems = (sem_a, sem_b)

    def start_gather(i):
        b = i % 2
        return pltpu.async_copy(
            table_hbm.at[idx_v.at[pl.ds(i * CHUNK, CHUNK)]], bufs[b], sems[b]
        )

    descs = {0: start_gather(0)}
    for i in range(N_CHUNKS):
        if i + 1 < N_CHUNKS:
            descs[i + 1] = start_gather(i + 1)
        descs[i].wait()
        pltpu.sync_copy(bufs[i % 2], out_hbm.at[pl.ds(base + i * CHUNK, CHUNK)])


TBLK = 2048
T_GRID = -(-NUM_EMB // TBLK)


def _transpose_body(src, dst):
    dst[...] = jnp.concatenate(
        [src[...].T, jnp.zeros((TBLK, 128 - DIM), jnp.float32)], axis=1
    )


_convert_table = pl.pallas_call(
    _transpose_body,
    grid=(T_GRID,),
    in_specs=[pl.BlockSpec((DIM, TBLK), lambda i: (0, i))],
    out_specs=pl.BlockSpec((TBLK, 128), lambda i: (i, 0)),
    out_shape=jax.ShapeDtypeStruct((NUM_EMB, 128), jnp.float32),
)


def kernel(indices, embedding):
    flat_idx = indices.reshape(-1).astype(jnp.int32) * 2
    emb_pad = _convert_table(embedding.T)
    emb_linear = emb_pad.reshape(2 * NUM_EMB, DIM)
    out = _gather_kernel(flat_idx, emb_linear)
    return out.reshape(indices.shape + (DIM,))


# speedup vs baseline: 1.3467x; 1.1773x over previous
import functools

import jax
import jax.numpy as jnp
from jax import lax
from jax.experimental import pallas as pl
from jax.experimental.pallas import tpu as pltpu
from jax.experimental.pallas import tpu_sc as plsc

NUM_EMB = 1000000
NUM_ROWS = 16384 * 26
DIM = 64
NC = 2
NS = 16
NW = NC * NS
ROWS_PER_W = NUM_ROWS // NW
CHUNK = 512
N_CHUNKS = ROWS_PER_W // CHUNK

_mesh = plsc.VectorSubcoreMesh(
    core_axis_name="c", subcore_axis_name="s", num_cores=NC, num_subcores=NS
)


@functools.partial(
    pl.kernel,
    out_type=jax.ShapeDtypeStruct((NUM_ROWS, DIM), jnp.float32),
    mesh=_mesh,
    scratch_types=[
        pltpu.VMEM((ROWS_PER_W,), jnp.int32),
        pltpu.VMEM((CHUNK, DIM), jnp.float32),
        pltpu.VMEM((CHUNK, DIM), jnp.float32),
        pltpu.SemaphoreType.DMA,
        pltpu.SemaphoreType.DMA,
    ],
    compiler_params=pltpu.CompilerParams(use_tc_tiling_on_sc=False),
)
def _gather_kernel(idx_hbm, table_hbm, out_hbm, idx_v, rows_a, rows_b, sem_a, sem_b):
    wid = lax.axis_index("s") * NC + lax.axis_index("c")
    base = wid * ROWS_PER_W
    pltpu.sync_copy(idx_hbm.at[pl.ds(base, ROWS_PER_W)], idx_v)

    bufs = (rows_a, rows_b)
    sems = (sem_a, sem_b)

    def start_gather(i):
        b = i % 2
        return pltpu.async_copy(
            table_hbm.at[idx_v.at[pl.ds(i * CHUNK, CHUNK)]], bufs[b], sems[b]
        )

    descs = {0: start_gather(0)}
    for i in range(N_CHUNKS):
        if i + 1 < N_CHUNKS:
            descs[i + 1] = start_gather(i + 1)
        descs[i].wait()
        pltpu.sync_copy(bufs[i % 2], out_hbm.at[pl.ds(base + i * CHUNK, CHUNK)])


TBLK = 4096
T_GRID = -(-NUM_EMB // TBLK)


def _transpose_body(src, dst):
    dst[...] = jnp.concatenate(
        [src[...].T, jnp.zeros((TBLK, 128 - DIM), jnp.float32)], axis=1
    )


_convert_table = pl.pallas_call(
    _transpose_body,
    grid=(T_GRID,),
    in_specs=[pl.BlockSpec((DIM, TBLK), lambda i: (0, i))],
    out_specs=pl.BlockSpec((TBLK, 128), lambda i: (i, 0)),
    out_shape=jax.ShapeDtypeStruct((NUM_EMB, 128), jnp.float32),
)


def kernel(indices, embedding):
    flat_idx = indices.reshape(-1).astype(jnp.int32) * 2
    emb_pad = _convert_table(embedding.T)
    emb_linear = emb_pad.reshape(2 * NUM_EMB, DIM)
    out = _gather_kernel(flat_idx, emb_linear)
    return out.reshape(indices.shape + (DIM,))


# speedup vs baseline: 1.4984x; 1.1126x over previous
import functools

import jax
import jax.numpy as jnp
from jax import lax
from jax.experimental import pallas as pl
from jax.experimental.pallas import tpu as pltpu
from jax.experimental.pallas import tpu_sc as plsc

NUM_EMB = 1000000
NUM_ROWS = 16384 * 26
DIM = 64
NC = 2
NS = 16
NW = NC * NS
ROWS_PER_W = NUM_ROWS // NW
CHUNK = 512
N_CHUNKS = ROWS_PER_W // CHUNK

_mesh = plsc.VectorSubcoreMesh(
    core_axis_name="c", subcore_axis_name="s", num_cores=NC, num_subcores=NS
)


@functools.partial(
    pl.kernel,
    out_type=jax.ShapeDtypeStruct((NUM_ROWS, DIM), jnp.float32),
    mesh=_mesh,
    scratch_types=[
        pltpu.VMEM((ROWS_PER_W,), jnp.int32),
        pltpu.VMEM((CHUNK, DIM), jnp.float32),
        pltpu.VMEM((CHUNK, DIM), jnp.float32),
        pltpu.SemaphoreType.DMA,
        pltpu.SemaphoreType.DMA,
    ],
    compiler_params=pltpu.CompilerParams(use_tc_tiling_on_sc=False),
)
def _gather_kernel(idx_hbm, table_hbm, out_hbm, idx_v, rows_a, rows_b, sem_a, sem_b):
    wid = lax.axis_index("s") * NC + lax.axis_index("c")
    base = wid * ROWS_PER_W
    pltpu.sync_copy(idx_hbm.at[pl.ds(base, ROWS_PER_W)], idx_v)

    bufs = (rows_a, rows_b)
    sems = (sem_a, sem_b)

    def start_gather(i):
        b = i % 2
        return pltpu.async_copy(
            table_hbm.at[idx_v.at[pl.ds(i * CHUNK, CHUNK)]], bufs[b], sems[b]
        )

    descs = {0: start_gather(0)}
    for i in range(N_CHUNKS):
        if i + 1 < N_CHUNKS:
            descs[i + 1] = start_gather(i + 1)
        descs[i].wait()
        pltpu.sync_copy(bufs[i % 2], out_hbm.at[pl.ds(base + i * CHUNK, CHUNK)])


TBLK = 8192
T_GRID = -(-NUM_EMB // TBLK)


def _transpose_body(src, dst):
    dst[...] = jnp.concatenate(
        [src[...].T, jnp.zeros((TBLK, 128 - DIM), jnp.float32)], axis=1
    )


_convert_table = pl.pallas_call(
    _transpose_body,
    grid=(T_GRID,),
    in_specs=[pl.BlockSpec((DIM, TBLK), lambda i: (0, i))],
    out_specs=pl.BlockSpec((TBLK, 128), lambda i: (i, 0)),
    out_shape=jax.ShapeDtypeStruct((NUM_EMB, 128), jnp.float32),
)


def kernel(indices, embedding):
    flat_idx = indices.reshape(-1).astype(jnp.int32) * 2
    emb_pad = _convert_table(embedding.T)
    emb_linear = emb_pad.reshape(2 * NUM_EMB, DIM)
    out = _gather_kernel(flat_idx, emb_linear)
    return out.reshape(indices.shape + (DIM,))
